# Initial kernel scaffold; baseline (speedup 1.0000x reference)
import jax
import jax.numpy as jnp
from jax.experimental import pallas as pl


def kernel(mod_indices, table, ln_gamma, ln_beta):
    raise NotImplementedError("write your pallas kernel here")


# speedup vs baseline: 2.0053x; 2.0053x over previous
import functools

import jax
import jax.numpy as jnp
from jax import lax
from jax.experimental import pallas as pl
from jax.experimental.pallas import tpu as pltpu
from jax.experimental.pallas import tpu_sc as plsc

NUM_MODS = 1000
EMBED_DIM = 128
BATCH = 16384

_INFO = plsc.get_sparse_core_info()
_NC, _NS, _L = _INFO.num_cores, _INFO.num_subcores, _INFO.num_lanes
_NW = _NC * _NS
_RPW = BATCH // _NW
_CHUNK = 128
_NCHUNK = _RPW // _CHUNK
_NVREG = EMBED_DIM // _L
_G = 4


def _rsqrt_vec(a):
    i = plsc.bitcast(a, jnp.int32)
    i = jnp.int32(0x5F3759DF) - (i >> 1)
    y = plsc.bitcast(i, jnp.float32)
    for _ in range(3):
        y = y * (1.5 - 0.5 * a * y * y)
    return y


def _body(idx_hbm, table_hbm, gamma_hbm, beta_hbm, out_hbm,
          idx_v, rows_v, g_v, b_v, sem):
    wid = lax.axis_index("s") * _NC + lax.axis_index("c")
    base = wid * _RPW

    for j in range(_NCHUNK):
        pltpu.sync_copy(idx_hbm.at[pl.ds(base + j * _CHUNK, _CHUNK)],
                        idx_v.at[j])
    pltpu.sync_copy(gamma_hbm, g_v)
    pltpu.sync_copy(beta_hbm, b_v)

    copies = [
        pltpu.async_copy(table_hbm.at[idx_v.at[j]],
                         rows_v.at[pl.ds(j * _CHUNK, _CHUNK)], sem)
        for j in range(_NCHUNK)
    ]
    for c in copies:
        c.wait()

    g = [g_v[pl.ds(k * _L, _L)] for k in range(_NVREG)]
    b = [b_v[pl.ds(k * _L, _L)] for k in range(_NVREG)]
    inv_n = jnp.float32(1.0 / EMBED_DIM)

    lane = lax.iota(jnp.int32, _L)
    perms = [lane ^ m for m in (1, 2, 4, 8)]

    dnums = lax.GatherDimensionNumbers(
        offset_dims=(), collapsed_slice_dims=(0,), start_index_map=(0,))

    def lane_sum(x):
        for p in perms:
            x = x + lax.gather(x, p[:, None], dnums, slice_sizes=(1,),
                               mode=lax.GatherScatterMode.PROMISE_IN_BOUNDS)
        return x

    def row_block(i, _):
        for u in range(_G):
            r = i * _G + u
            v = [rows_v[r, pl.ds(k * _L, _L)] for k in range(_NVREG)]
            s01, s23 = v[0] + v[1], v[2] + v[3]
            s45, s67 = v[4] + v[5], v[6] + v[7]
            s = (s01 + s23) + (s45 + s67)
            q01, q23 = v[0] * v[0] + v[1] * v[1], v[2] * v[2] + v[3] * v[3]
            q45, q67 = v[4] * v[4] + v[5] * v[5], v[6] * v[6] + v[7] * v[7]
            q = (q01 + q23) + (q45 + q67)
            mean = lane_sum(s) * inv_n
            av = jnp.maximum(lane_sum(q) * inv_n - mean * mean, 0.0) + 1e-5
            rinv = _rsqrt_vec(av)
            for k in range(_NVREG):
                rows_v[r, pl.ds(k * _L, _L)] = (v[k] - mean) * rinv * g[k] + b[k]
        return _

    lax.fori_loop(0, _RPW // _G, row_block, 0)

    pltpu.sync_copy(rows_v, out_hbm.at[pl.ds(base, _RPW)])


@functools.partial(jax.jit)
def _run(mod_indices, table, ln_gamma, ln_beta):
    mesh = plsc.VectorSubcoreMesh(core_axis_name="c", subcore_axis_name="s")
    k = pl.kernel(
        _body,
        out_type=jax.ShapeDtypeStruct((BATCH, EMBED_DIM), jnp.float32),
        mesh=mesh,
        scratch_types=[
            pltpu.VMEM((_NCHUNK, _CHUNK), jnp.int32),
            pltpu.VMEM((_RPW, EMBED_DIM), jnp.float32),
            pltpu.VMEM((EMBED_DIM,), jnp.float32),
            pltpu.VMEM((EMBED_DIM,), jnp.float32),
            pltpu.SemaphoreType.DMA,
        ],
        compiler_params=pltpu.CompilerParams(needs_layout_passes=False),
    )
    return k(mod_indices, table, ln_gamma, ln_beta)


def kernel(mod_indices, table, ln_gamma, ln_beta):
    idx = mod_indices.astype(jnp.int32)
    return _run(idx, table, ln_gamma, ln_beta)


# speedup vs baseline: 2.3683x; 1.1810x over previous
import functools

import jax
import jax.numpy as jnp
from jax import lax
from jax.experimental import pallas as pl
from jax.experimental.pallas import tpu as pltpu
from jax.experimental.pallas import tpu_sc as plsc

NUM_MODS = 1000
EMBED_DIM = 128
BATCH = 16384

_INFO = plsc.get_sparse_core_info()
_NC, _NS, _L = _INFO.num_cores, _INFO.num_subcores, _INFO.num_lanes
_NW = _NC * _NS
_RPW = BATCH // _NW
_CHUNK = 128
_NCHUNK = _RPW // _CHUNK
_NVREG = EMBED_DIM // _L
_G = 4


def _rsqrt_vec(a):
    i = plsc.bitcast(a, jnp.int32)
    i = jnp.int32(0x5F3759DF) - (i >> 1)
    y = plsc.bitcast(i, jnp.float32)
    for _ in range(3):
        y = y * (1.5 - 0.5 * a * y * y)
    return y


def _body(idx_hbm, table_hbm, gamma_hbm, beta_hbm, out_hbm,
          idx_v, rows_v, g_v, b_v, sem_g0, sem_g1, sem_out, sem_misc):
    wid = lax.axis_index("s") * _NC + lax.axis_index("c")
    base = wid * _RPW
    sems_g = (sem_g0, sem_g1)

    idx_cp = pltpu.async_copy(idx_hbm.at[pl.ds(wid * _NCHUNK, _NCHUNK)],
                              idx_v, sem_misc)
    g_cp = pltpu.async_copy(gamma_hbm, g_v, sem_misc)
    b_cp = pltpu.async_copy(beta_hbm, b_v, sem_misc)
    idx_cp.wait()
    g_cp.wait()
    b_cp.wait()

    def fire_gather(j):
        return pltpu.async_copy(table_hbm.at[idx_v.at[j]],
                                rows_v.at[pl.ds(j * _CHUNK, _CHUNK)],
                                sems_g[j % 2])

    g = [g_v[pl.ds(k * _L, _L)] for k in range(_NVREG)]
    b = [b_v[pl.ds(k * _L, _L)] for k in range(_NVREG)]
    inv_n = jnp.float32(1.0 / EMBED_DIM)

    lane = lax.iota(jnp.int32, _L)
    perms = [lane ^ m for m in (1, 2, 4, 8)]
    dnums = lax.GatherDimensionNumbers(
        offset_dims=(), collapsed_slice_dims=(0,), start_index_map=(0,))

    def lane_sum(x):
        for p in perms:
            x = x + lax.gather(x, p[:, None], dnums, slice_sizes=(1,),
                               mode=lax.GatherScatterMode.PROMISE_IN_BOUNDS)
        return x

    def make_row_block(chunk_base):
        def row_block(i, _):
            for u in range(_G):
                r = chunk_base + i * _G + u
                v = [rows_v[r, pl.ds(k * _L, _L)] for k in range(_NVREG)]
                s01, s23 = v[0] + v[1], v[2] + v[3]
                s45, s67 = v[4] + v[5], v[6] + v[7]
                s = (s01 + s23) + (s45 + s67)
                q01 = v[0] * v[0] + v[1] * v[1]
                q23 = v[2] * v[2] + v[3] * v[3]
                q45 = v[4] * v[4] + v[5] * v[5]
                q67 = v[6] * v[6] + v[7] * v[7]
                q = (q01 + q23) + (q45 + q67)
                mean = lane_sum(s) * inv_n
                av = lane_sum(q) * inv_n - mean * mean + 1e-5
                rinv = _rsqrt_vec(av)
                for k in range(_NVREG):
                    rows_v[r, pl.ds(k * _L, _L)] = \
                        (v[k] - mean) * rinv * g[k] + b[k]
            return _
        return row_block

    gathers = [fire_gather(0)] + [None] * (_NCHUNK - 1)
    writes = []
    for c in range(_NCHUNK):
        gathers[c].wait()
        if c + 1 < _NCHUNK:
            gathers[c + 1] = fire_gather(c + 1)
        lax.fori_loop(0, _CHUNK // _G, make_row_block(c * _CHUNK), 0)
        writes.append(
            pltpu.async_copy(rows_v.at[pl.ds(c * _CHUNK, _CHUNK)],
                             out_hbm.at[pl.ds(base + c * _CHUNK, _CHUNK)],
                             sem_out))
    for w in writes:
        w.wait()


@functools.partial(jax.jit)
def _run(idx2d, table, ln_gamma, ln_beta):
    mesh = plsc.VectorSubcoreMesh(core_axis_name="c", subcore_axis_name="s")
    k = pl.kernel(
        _body,
        out_type=jax.ShapeDtypeStruct((BATCH, EMBED_DIM), jnp.float32),
        mesh=mesh,
        scratch_types=[
            pltpu.VMEM((_NCHUNK, _CHUNK), jnp.int32),
            pltpu.VMEM((_RPW, EMBED_DIM), jnp.float32),
            pltpu.VMEM((EMBED_DIM,), jnp.float32),
            pltpu.VMEM((EMBED_DIM,), jnp.float32),
            pltpu.SemaphoreType.DMA,
            pltpu.SemaphoreType.DMA,
            pltpu.SemaphoreType.DMA,
            pltpu.SemaphoreType.DMA,
        ],
        compiler_params=pltpu.CompilerParams(needs_layout_passes=False),
    )
    return k(idx2d, table, ln_gamma, ln_beta)


def kernel(mod_indices, table, ln_gamma, ln_beta):
    idx2d = mod_indices.astype(jnp.int32).reshape(BATCH // _CHUNK, _CHUNK)
    return _run(idx2d, table, ln_gamma, ln_beta)


# speedup vs baseline: 2.5648x; 1.0830x over previous
import functools

import jax
import jax.numpy as jnp
from jax import lax
from jax.experimental import pallas as pl
from jax.experimental.pallas import tpu as pltpu
from jax.experimental.pallas import tpu_sc as plsc

NUM_MODS = 1000
EMBED_DIM = 128
BATCH = 16384

_INFO = plsc.get_sparse_core_info()
_NC, _NS, _L = _INFO.num_cores, _INFO.num_subcores, _INFO.num_lanes
_NW = _NC * _NS
_RPW = BATCH // _NW
_CHUNK = 128
_NCHUNK = _RPW // _CHUNK
_NVREG = EMBED_DIM // _L
_G = 4
_TROWS = 64
_TPAD = _TROWS * _NS
_STAGE = _RPW - _TROWS
_HALF = BATCH // _NC


def _rsqrt_vec(a):
    i = plsc.bitcast(a, jnp.int32)
    i = jnp.int32(0x5F3759DF) - (i >> 1)
    y = plsc.bitcast(i, jnp.float32)
    for _ in range(3):
        y = y * (1.5 - 0.5 * a * y * y)
    return y


def _body(idx_hbm, table_hbm, gamma_hbm, beta_hbm, out_hbm,
          idx_v, rows_v, tbl_v, g_v, b_v, sem_g0, sem_g1, sem_out, sem_misc):
    c = lax.axis_index("c")
    s = lax.axis_index("s")
    blk = c * _NS + s
    base = blk * _RPW
    sc_off = c * _HALF + _STAGE
    sems_g = (sem_g0, sem_g1)

    t_cp = pltpu.async_copy(table_hbm.at[pl.ds(s * _TROWS, _TROWS)],
                            tbl_v, sem_misc)
    i_cp = pltpu.async_copy(idx_hbm.at[pl.ds(blk * _NCHUNK, _NCHUNK)],
                            idx_v, sem_misc)
    g_cp = pltpu.async_copy(gamma_hbm, g_v, sem_misc)
    b_cp = pltpu.async_copy(beta_hbm, b_v, sem_misc)
    t_cp.wait()
    i_cp.wait()
    g_cp.wait()
    b_cp.wait()

    g = [g_v[pl.ds(k * _L, _L)] for k in range(_NVREG)]
    b = [b_v[pl.ds(k * _L, _L)] for k in range(_NVREG)]
    inv_n = jnp.float32(1.0 / EMBED_DIM)

    lane = lax.iota(jnp.int32, _L)
    perms = [lane ^ m for m in (1, 2, 4, 8)]
    dnums = lax.GatherDimensionNumbers(
        offset_dims=(), collapsed_slice_dims=(0,), start_index_map=(0,))

    def lane_sum(x):
        for p in perms:
            x = x + lax.gather(x, p[:, None], dnums, slice_sizes=(1,),
                               mode=lax.GatherScatterMode.PROMISE_IN_BOUNDS)
        return x

    def row_block(i, _):
        for u in range(_G):
            r = i * _G + u
            v = [tbl_v[r, pl.ds(k * _L, _L)] for k in range(_NVREG)]
            s01, s23 = v[0] + v[1], v[2] + v[3]
            s45, s67 = v[4] + v[5], v[6] + v[7]
            sm = (s01 + s23) + (s45 + s67)
            q01 = v[0] * v[0] + v[1] * v[1]
            q23 = v[2] * v[2] + v[3] * v[3]
            q45 = v[4] * v[4] + v[5] * v[5]
            q67 = v[6] * v[6] + v[7] * v[7]
            q = (q01 + q23) + (q45 + q67)
            mean = lane_sum(sm) * inv_n
            av = lane_sum(q) * inv_n - mean * mean + 1e-5
            rinv = _rsqrt_vec(av)
            for k in range(_NVREG):
                tbl_v[r, pl.ds(k * _L, _L)] = \
                    (v[k] - mean) * rinv * g[k] + b[k]
        return _

    lax.fori_loop(0, _TROWS // _G, row_block, 0)

    stage_cp = pltpu.async_copy(
        tbl_v, out_hbm.at[pl.ds(base + _STAGE, _TROWS)], sem_out)

    for j in range(_NCHUNK):
        for k in range(_NVREG):
            t = idx_v[j, pl.ds(k * _L, _L)]
            r = ((t & jnp.int32(~63)) << 3) + (t & jnp.int32(63)) + sc_off
            idx_v[j, pl.ds(k * _L, _L)] = r

    stage_cp.wait()
    plsc.subcore_barrier()

    def fire_gather(j):
        return pltpu.async_copy(out_hbm.at[idx_v.at[j]],
                                rows_v.at[pl.ds(j * _CHUNK, _CHUNK)],
                                sems_g[j % 2])

    gathers = [fire_gather(0), fire_gather(1)] + [None] * (_NCHUNK - 2)
    writes = []
    for j in range(_NCHUNK):
        gathers[j].wait()
        if j + 2 < _NCHUNK:
            gathers[j + 2] = fire_gather(j + 2)
        if j < _NCHUNK - 1:
            writes.append(
                pltpu.async_copy(rows_v.at[pl.ds(j * _CHUNK, _CHUNK)],
                                 out_hbm.at[pl.ds(base + j * _CHUNK, _CHUNK)],
                                 sem_out))
    plsc.subcore_barrier()
    writes.append(
        pltpu.async_copy(rows_v.at[pl.ds((_NCHUNK - 1) * _CHUNK, _CHUNK)],
                         out_hbm.at[pl.ds(base + (_NCHUNK - 1) * _CHUNK,
                                          _CHUNK)],
                         sem_out))
    for w in writes:
        w.wait()


@functools.partial(jax.jit)
def _run(idx2d, table_p, ln_gamma, ln_beta):
    mesh = plsc.VectorSubcoreMesh(core_axis_name="c", subcore_axis_name="s")
    k = pl.kernel(
        _body,
        out_type=jax.ShapeDtypeStruct((BATCH, EMBED_DIM), jnp.float32),
        mesh=mesh,
        scratch_types=[
            pltpu.VMEM((_NCHUNK, _CHUNK), jnp.int32),
            pltpu.VMEM((_RPW, EMBED_DIM), jnp.float32),
            pltpu.VMEM((_TROWS, EMBED_DIM), jnp.float32),
            pltpu.VMEM((EMBED_DIM,), jnp.float32),
            pltpu.VMEM((EMBED_DIM,), jnp.float32),
            pltpu.SemaphoreType.DMA,
            pltpu.SemaphoreType.DMA,
            pltpu.SemaphoreType.DMA,
            pltpu.SemaphoreType.DMA,
        ],
        compiler_params=pltpu.CompilerParams(needs_layout_passes=False),
    )
    return k(idx2d, table_p, ln_gamma, ln_beta)


def kernel(mod_indices, table, ln_gamma, ln_beta):
    idx2d = mod_indices.astype(jnp.int32).reshape(BATCH // _CHUNK, _CHUNK)
    table_p = jnp.pad(table, ((0, _TPAD - NUM_MODS), (0, 0)))
    return _run(idx2d, table_p, ln_gamma, ln_beta)


# speedup vs baseline: 3.0305x; 1.1816x over previous
import functools

import jax
import jax.numpy as jnp
from jax import lax
from jax.experimental import pallas as pl
from jax.experimental.pallas import tpu as pltpu
from jax.experimental.pallas import tpu_sc as plsc

NUM_MODS = 1000
EMBED_DIM = 128
BATCH = 16384

_INFO = plsc.get_sparse_core_info()
_NC, _NS, _L = _INFO.num_cores, _INFO.num_subcores, _INFO.num_lanes
_NW = _NC * _NS
_RPW = BATCH // _NW
_CHUNK = 128
_NCHUNK = _RPW // _CHUNK
_NVREG = EMBED_DIM // _L
_G = 4
_TROWS = 64
_TLAST = NUM_MODS - _TROWS


def _rsqrt_vec(a):
    i = plsc.bitcast(a, jnp.int32)
    i = jnp.int32(0x5F3759DF) - (i >> 1)
    y = plsc.bitcast(i, jnp.float32)
    for _ in range(3):
        y = y * (1.5 - 0.5 * a * y * y)
    return y


def _body(idx_hbm, table_hbm, gamma_hbm, beta_hbm, out_hbm,
          idx_v, rows_v, tbl_v, g_v, b_v, ntab_sh,
          sem_g0, sem_g1, sem_out, sem_misc):
    c = lax.axis_index("c")
    s = lax.axis_index("s")
    blk = c * _NS + s
    base = blk * _RPW
    sems_g = (sem_g0, sem_g1)
    trow = jnp.minimum(s * _TROWS, _TLAST)

    t_cp = pltpu.async_copy(table_hbm.at[pl.ds(trow, _TROWS)],
                            tbl_v, sem_misc)
    i_cp = pltpu.async_copy(idx_hbm.at[pl.ds(blk * _NCHUNK, _NCHUNK)],
                            idx_v, sem_misc)
    g_cp = pltpu.async_copy(gamma_hbm, g_v, sem_misc)
    b_cp = pltpu.async_copy(beta_hbm, b_v, sem_misc)
    t_cp.wait()
    i_cp.wait()
    g_cp.wait()
    b_cp.wait()

    g = [g_v[pl.ds(k * _L, _L)] for k in range(_NVREG)]
    b = [b_v[pl.ds(k * _L, _L)] for k in range(_NVREG)]
    inv_n = jnp.float32(1.0 / EMBED_DIM)

    lane = lax.iota(jnp.int32, _L)
    perms = [lane ^ m for m in (1, 2, 4, 8)]
    dnums = lax.GatherDimensionNumbers(
        offset_dims=(), collapsed_slice_dims=(0,), start_index_map=(0,))

    def lane_sum(x):
        for p in perms:
            x = x + lax.gather(x, p[:, None], dnums, slice_sizes=(1,),
                               mode=lax.GatherScatterMode.PROMISE_IN_BOUNDS)
        return x

    def row_block(i, _):
        for u in range(_G):
            r = i * _G + u
            v = [tbl_v[r, pl.ds(k * _L, _L)] for k in range(_NVREG)]
            s01, s23 = v[0] + v[1], v[2] + v[3]
            s45, s67 = v[4] + v[5], v[6] + v[7]
            sm = (s01 + s23) + (s45 + s67)
            q01 = v[0] * v[0] + v[1] * v[1]
            q23 = v[2] * v[2] + v[3] * v[3]
            q45 = v[4] * v[4] + v[5] * v[5]
            q67 = v[6] * v[6] + v[7] * v[7]
            q = (q01 + q23) + (q45 + q67)
            mean = lane_sum(sm) * inv_n
            av = lane_sum(q) * inv_n - mean * mean + 1e-5
            rinv = _rsqrt_vec(av)
            for k in range(_NVREG):
                tbl_v[r, pl.ds(k * _L, _L)] = \
                    (v[k] - mean) * rinv * g[k] + b[k]
        return _

    lax.fori_loop(0, _TROWS // _G, row_block, 0)

    pltpu.sync_copy(tbl_v, ntab_sh.at[pl.ds(trow, _TROWS)])
    plsc.subcore_barrier()

    def fire_gather(j):
        return pltpu.async_copy(ntab_sh.at[idx_v.at[j]],
                                rows_v.at[pl.ds(j * _CHUNK, _CHUNK)],
                                sems_g[j % 2])

    gathers = [fire_gather(0), fire_gather(1)] + [None] * (_NCHUNK - 2)
    writes = []
    for j in range(_NCHUNK):
        gathers[j].wait()
        if j + 2 < _NCHUNK:
            gathers[j + 2] = fire_gather(j + 2)
        writes.append(
            pltpu.async_copy(rows_v.at[pl.ds(j * _CHUNK, _CHUNK)],
                             out_hbm.at[pl.ds(base + j * _CHUNK, _CHUNK)],
                             sem_out))
    for w in writes:
        w.wait()


@functools.partial(jax.jit)
def _run(idx2d, table, ln_gamma, ln_beta):
    mesh = plsc.VectorSubcoreMesh(core_axis_name="c", subcore_axis_name="s")
    k = pl.kernel(
        _body,
        out_type=jax.ShapeDtypeStruct((BATCH, EMBED_DIM), jnp.float32),
        mesh=mesh,
        scratch_types=[
            pltpu.VMEM((_NCHUNK, _CHUNK), jnp.int32),
            pltpu.VMEM((_RPW, EMBED_DIM), jnp.float32),
            pltpu.VMEM((_TROWS, EMBED_DIM), jnp.float32),
            pltpu.VMEM((EMBED_DIM,), jnp.float32),
            pltpu.VMEM((EMBED_DIM,), jnp.float32),
            pltpu.VMEM_SHARED((NUM_MODS, EMBED_DIM), jnp.float32),
            pltpu.SemaphoreType.DMA,
            pltpu.SemaphoreType.DMA,
            pltpu.SemaphoreType.DMA,
            pltpu.SemaphoreType.DMA,
        ],
        compiler_params=pltpu.CompilerParams(needs_layout_passes=False),
    )
    return k(idx2d, table, ln_gamma, ln_beta)


def kernel(mod_indices, table, ln_gamma, ln_beta):
    idx2d = mod_indices.astype(jnp.int32).reshape(BATCH // _CHUNK, _CHUNK)
    return _run(idx2d, table, ln_gamma, ln_beta)
